# baseline (speedup 1.0000x reference)
import functools
import jax
import jax.numpy as jnp
from jax import lax
from jax.experimental import pallas as pl
from jax.experimental.pallas import tpu as pltpu
from jax.experimental.pallas import tpu_sc as plsc

E = 8
K = 2
H = 768
INTER = 768
ALPHA = 1.702
LIMIT = 7.0

S = 2048
NSLOT = S * K
CB = 256
SB = 1024
NW = 32
TPW = S // NW


def _router_body(xb_ref, wr_ref, rb_ref, idx_ref, w_ref, starts_ref):
    logits = jax.lax.dot_general(
        xb_ref[...], wr_ref[...], (((1,), (1,)), ((), ())),
        preferred_element_type=jnp.float32) + rb_ref[...]
    m = jnp.max(logits, axis=-1, keepdims=True)
    ex = jnp.exp(logits - m)
    p = ex / jnp.sum(ex, axis=-1, keepdims=True)
    lanes = jax.lax.broadcasted_iota(jnp.int32, (S, E), 1)
    a1 = jnp.argmax(p, axis=-1)
    oh1 = (lanes == a1[:, None]).astype(jnp.float32)
    p2 = jnp.where(oh1 > 0, -jnp.inf, p)
    a2 = jnp.argmax(p2, axis=-1)
    oh2 = (lanes == a2[:, None]).astype(jnp.float32)
    w0 = jnp.max(p, axis=-1)
    w1 = jnp.max(p2, axis=-1)

    cnt0 = jnp.sum(oh1, axis=0, keepdims=True)
    cnt1 = jnp.sum(oh2, axis=0, keepdims=True)
    counts = cnt0 + cnt1
    r8 = jax.lax.broadcasted_iota(jnp.int32, (E, E), 0)
    c8 = jax.lax.broadcasted_iota(jnp.int32, (E, E), 1)
    tri = (r8 < c8).astype(jnp.float32)
    starts_f = jnp.dot(counts, tri, preferred_element_type=jnp.float32,
                       precision=jax.lax.Precision.HIGHEST)

    rS = jax.lax.broadcasted_iota(jnp.int32, (S, S), 0)
    cS = jax.lax.broadcasted_iota(jnp.int32, (S, S), 1)
    tri_s = (cS < rS).astype(jnp.bfloat16)
    cs0 = jnp.dot(tri_s, oh1.astype(jnp.bfloat16), preferred_element_type=jnp.float32)
    cs1 = jnp.dot(tri_s, oh2.astype(jnp.bfloat16), preferred_element_type=jnp.float32)
    pos0 = jnp.sum(oh1 * (starts_f + cs0), axis=1)
    pos1 = jnp.sum(oh2 * (starts_f + cnt0 + cs1), axis=1)
    pos0_i = pos0.astype(jnp.int32)
    pos1_i = pos1.astype(jnp.int32)

    idx_ref[0, :] = pos0_i
    idx_ref[1, :] = pos1_i
    w_ref[0, :] = w0
    w_ref[1, :] = w1

    lane16 = jax.lax.broadcasted_iota(jnp.int32, (1, 16), 1)
    starts16 = jnp.concatenate(
        [starts_f.astype(jnp.int32), jnp.full((1, 8), NSLOT, jnp.int32)], axis=1)
    starts_ref[...] = jnp.where(lane16 < 9, starts16, NSLOT)


def _run_router(xb, wr, rb):
    return pl.pallas_call(
        _router_body,
        grid=(1,),
        in_specs=[
            pl.BlockSpec((S, H), lambda i: (0, 0)),
            pl.BlockSpec((E, H), lambda i: (0, 0)),
            pl.BlockSpec((1, E), lambda i: (0, 0)),
        ],
        out_specs=[
            pl.BlockSpec((2, S), lambda i: (0, 0)),
            pl.BlockSpec((2, S), lambda i: (0, 0)),
            pl.BlockSpec((1, 16), lambda i: (0, 0)),
        ],
        out_shape=[
            jax.ShapeDtypeStruct((2, S), jnp.int32),
            jax.ShapeDtypeStruct((2, S), jnp.float32),
            jax.ShapeDtypeStruct((1, 16), jnp.int32),
        ],
    )(xb, wr, rb)


HW = H // 2

def _dispatch_sc(xi, pos0, pos1, w0, w1):
    mesh = plsc.VectorSubcoreMesh(core_axis_name="c", subcore_axis_name="s")

    @functools.partial(
        pl.kernel, mesh=mesh,
        out_type=[
            jax.ShapeDtypeStruct((NSLOT, HW), jnp.int32),
            jax.ShapeDtypeStruct((NSLOT,), jnp.float32),
        ],
        scratch_types=[
            pltpu.VMEM((TPW,), jnp.int32),
            pltpu.VMEM((TPW,), jnp.int32),
            pltpu.VMEM((TPW,), jnp.float32),
            pltpu.VMEM((TPW,), jnp.float32),
            pltpu.VMEM((TPW, HW), jnp.int32),
            pltpu.SemaphoreType.DMA,
        ],
    )
    def k(x_hbm, p0_hbm, p1_hbm, w0_hbm, w1_hbm, xs_hbm, ws_hbm,
          i0_v, i1_v, w0_v, w1_v, rows_v, sem):
        wid = lax.axis_index("s") * 2 + lax.axis_index("c")
        base = wid * TPW
        loads = [
            pltpu.async_copy(p0_hbm.at[pl.ds(base, TPW)], i0_v, sem),
            pltpu.async_copy(p1_hbm.at[pl.ds(base, TPW)], i1_v, sem),
            pltpu.async_copy(w0_hbm.at[pl.ds(base, TPW)], w0_v, sem),
            pltpu.async_copy(w1_hbm.at[pl.ds(base, TPW)], w1_v, sem),
            pltpu.async_copy(x_hbm.at[pl.ds(base, TPW)], rows_v, sem),
        ]
        for cp in loads:
            cp.wait()
        stores = [
            pltpu.async_copy(rows_v, xs_hbm.at[i0_v], sem),
            pltpu.async_copy(rows_v, xs_hbm.at[i1_v], sem),
            pltpu.async_copy(w0_v, ws_hbm.at[i0_v], sem),
            pltpu.async_copy(w1_v, ws_hbm.at[i1_v], sem),
        ]
        for cp in stores:
            cp.wait()

    return k(xi, pos0, pos1, w0, w1)


def _gmm_body(starts_ref, xs_ref, ws_ref, wgu_ref, bgu_ref, wd_ref, bd_ref,
              ys_ref):
    b = pl.program_id(0)
    row0 = b * CB
    xb = xs_ref[...]
    riota = jax.lax.broadcasted_iota(jnp.int32, (CB, 1), 0)
    ys_ref[...] = jnp.zeros_like(ys_ref)
    for e in range(E):
        lo = jnp.maximum(starts_ref[0, e], row0)
        hi = jnp.minimum(starts_ref[0, e + 1], row0 + CB)

        @pl.when(hi > lo)
        def _(e=e, lo=lo, hi=hi):
            gu = jnp.dot(xb, wgu_ref[e], preferred_element_type=jnp.float32)
            gu = gu + bgu_ref[e, 0][None, :]
            gate = jnp.minimum(gu[:, :INTER], LIMIT)
            up = jnp.clip(gu[:, INTER:], -LIMIT, LIMIT)
            glu = gate * jax.nn.sigmoid(gate * ALPHA)
            act = ((up + 1.0) * glu).astype(jnp.bfloat16)
            dn = jnp.dot(act, wd_ref[e], preferred_element_type=jnp.float32)
            dn = dn + bd_ref[e, 0][None, :]
            msk = (riota >= lo - row0) & (riota < hi - row0)
            ys_ref[...] += jnp.where(msk, dn, 0.0)
    ys_ref[...] *= ws_ref[...]


def _run_gmm(starts, xs, ws, wgu, bgu, wd, bd):
    return pl.pallas_call(
        _gmm_body,
        grid=(NSLOT // CB,),
        in_specs=[
            pl.BlockSpec(memory_space=pltpu.SMEM),
            pl.BlockSpec((CB, H), lambda b: (b, 0)),
            pl.BlockSpec((CB, 1), lambda b: (b, 0)),
            pl.BlockSpec((E, H, 2 * INTER), lambda b: (0, 0, 0)),
            pl.BlockSpec((E, 1, 2 * INTER), lambda b: (0, 0, 0)),
            pl.BlockSpec((E, INTER, H), lambda b: (0, 0, 0)),
            pl.BlockSpec((E, 1, H), lambda b: (0, 0, 0)),
        ],
        out_specs=pl.BlockSpec((CB, H), lambda b: (b, 0)),
        out_shape=jax.ShapeDtypeStruct((NSLOT, H), jnp.float32),
    )(starts, xs, ws, wgu, bgu, wd, bd)


def _combine_sc(ys, pos0, pos1):
    mesh = plsc.VectorSubcoreMesh(core_axis_name="c", subcore_axis_name="s")

    @functools.partial(
        pl.kernel, mesh=mesh,
        out_type=jax.ShapeDtypeStruct((S, H), jnp.float32),
        scratch_types=[
            pltpu.VMEM((TPW,), jnp.int32),
            pltpu.VMEM((TPW,), jnp.int32),
            pltpu.VMEM((TPW, H), jnp.float32),
            pltpu.VMEM((TPW, H), jnp.float32),
            pltpu.SemaphoreType.DMA,
        ],
    )
    def k(ys_hbm, p0_hbm, p1_hbm, out_hbm, i0_v, i1_v, bufa, bufb, sem):
        wid = lax.axis_index("s") * 2 + lax.axis_index("c")
        base = wid * TPW
        l0 = pltpu.async_copy(p0_hbm.at[pl.ds(base, TPW)], i0_v, sem)
        l1 = pltpu.async_copy(p1_hbm.at[pl.ds(base, TPW)], i1_v, sem)
        l0.wait()
        l1.wait()
        g0 = pltpu.async_copy(ys_hbm.at[i0_v], bufa, sem)
        g1 = pltpu.async_copy(ys_hbm.at[i1_v], bufb, sem)
        g0.wait()
        g1.wait()

        def addrow(r, _):
            for cc in range(H // 16):
                sl = pl.ds(cc * 16, 16)
                bufa[r, sl] += bufb[r, sl]
            return 0
        lax.fori_loop(0, TPW, addrow, 0)
        pltpu.sync_copy(bufa, out_hbm.at[pl.ds(base, TPW)])

    return k(ys, pos0, pos1)


def kernel(hidden_states, router_weight, router_bias, gate_up_proj, gate_up_bias,
           down_proj, down_bias):
    B = hidden_states.shape[0]
    xb = hidden_states.reshape(S, H).astype(jnp.bfloat16)
    xi = lax.bitcast_convert_type(xb.reshape(S, HW, 2), jnp.int32)
    wr = router_weight.astype(jnp.bfloat16)
    rb = router_bias.reshape(1, E)
    wgu = gate_up_proj.astype(jnp.bfloat16)
    wd = down_proj.astype(jnp.bfloat16)
    bgu = gate_up_bias.reshape(E, 1, 2 * INTER)
    bd = down_bias.reshape(E, 1, H)

    idx2, w2, starts = _run_router(xb, wr, rb)
    pos0, pos1 = idx2[0], idx2[1]
    xs_i, ws = _dispatch_sc(xi, pos0, pos1, w2[0], w2[1])
    xs_bf = lax.bitcast_convert_type(xs_i, jnp.bfloat16).reshape(NSLOT, H)
    ys = _run_gmm(starts, xs_bf, ws.reshape(NSLOT, 1), wgu, bgu, wd, bd)
    out = _combine_sc(ys, pos0, pos1)
    return out.reshape(B, S, H)


# speedup vs baseline: 1.6361x; 1.6361x over previous
import functools
import jax
import jax.numpy as jnp
from jax import lax
from jax.experimental import pallas as pl
from jax.experimental.pallas import tpu as pltpu
from jax.experimental.pallas import tpu_sc as plsc

E = 8
K = 2
H = 768
INTER = 768
ALPHA = 1.702
LIMIT = 7.0

S = 2048
NSLOT = S * K
CB = 256
SB = 1024
NW = 32
TPW = S // NW


def _router_body(xb_ref, wr_ref, rb_ref, idx_ref, w_ref, starts_ref):
    logits = jax.lax.dot_general(
        xb_ref[...], wr_ref[...], (((1,), (1,)), ((), ())),
        preferred_element_type=jnp.float32) + rb_ref[...]
    m = jnp.max(logits, axis=-1, keepdims=True)
    ex = jnp.exp(logits - m)
    p = ex / jnp.sum(ex, axis=-1, keepdims=True)
    lanes = jax.lax.broadcasted_iota(jnp.int32, (S, E), 1)
    a1 = jnp.argmax(p, axis=-1)
    oh1 = (lanes == a1[:, None]).astype(jnp.float32)
    p2 = jnp.where(oh1 > 0, -jnp.inf, p)
    a2 = jnp.argmax(p2, axis=-1)
    oh2 = (lanes == a2[:, None]).astype(jnp.float32)
    w0 = jnp.max(p, axis=-1)
    w1 = jnp.max(p2, axis=-1)

    cnt0 = jnp.sum(oh1, axis=0, keepdims=True)
    cnt1 = jnp.sum(oh2, axis=0, keepdims=True)
    counts = cnt0 + cnt1
    r8 = jax.lax.broadcasted_iota(jnp.int32, (E, E), 0)
    c8 = jax.lax.broadcasted_iota(jnp.int32, (E, E), 1)
    tri = (r8 < c8).astype(jnp.float32)
    starts_f = jnp.dot(counts, tri, preferred_element_type=jnp.float32,
                       precision=jax.lax.Precision.HIGHEST)

    rS = jax.lax.broadcasted_iota(jnp.int32, (S, S), 0)
    cS = jax.lax.broadcasted_iota(jnp.int32, (S, S), 1)
    tri_s = (cS < rS).astype(jnp.bfloat16)
    cs0 = jnp.dot(tri_s, oh1.astype(jnp.bfloat16), preferred_element_type=jnp.float32)
    cs1 = jnp.dot(tri_s, oh2.astype(jnp.bfloat16), preferred_element_type=jnp.float32)
    pos0 = jnp.sum(oh1 * (starts_f + cs0), axis=1)
    pos1 = jnp.sum(oh2 * (starts_f + cnt0 + cs1), axis=1)
    pos0_i = pos0.astype(jnp.int32)
    pos1_i = pos1.astype(jnp.int32)

    idx_ref[0, :] = pos0_i
    idx_ref[1, :] = pos1_i
    w_ref[0, :] = w0
    w_ref[1, :] = w1

    lane16 = jax.lax.broadcasted_iota(jnp.int32, (1, 16), 1)
    starts16 = jnp.concatenate(
        [starts_f.astype(jnp.int32), jnp.full((1, 8), NSLOT, jnp.int32)], axis=1)
    starts_ref[...] = jnp.where(lane16 < 9, starts16, NSLOT)


def _run_router(xb, wr, rb):
    return pl.pallas_call(
        _router_body,
        grid=(1,),
        in_specs=[
            pl.BlockSpec((S, H), lambda i: (0, 0)),
            pl.BlockSpec((E, H), lambda i: (0, 0)),
            pl.BlockSpec((1, E), lambda i: (0, 0)),
        ],
        out_specs=[
            pl.BlockSpec((2, S), lambda i: (0, 0)),
            pl.BlockSpec((2, S), lambda i: (0, 0)),
            pl.BlockSpec((1, 16), lambda i: (0, 0)),
        ],
        out_shape=[
            jax.ShapeDtypeStruct((2, S), jnp.int32),
            jax.ShapeDtypeStruct((2, S), jnp.float32),
            jax.ShapeDtypeStruct((1, 16), jnp.int32),
        ],
    )(xb, wr, rb)


def _dispatch_sc(x, pos0, pos1, w0, w1):
    mesh = plsc.VectorSubcoreMesh(core_axis_name="c", subcore_axis_name="s")

    @functools.partial(
        pl.kernel, mesh=mesh,
        out_type=[
            jax.ShapeDtypeStruct((NSLOT, H), jnp.float32),
            jax.ShapeDtypeStruct((NSLOT,), jnp.float32),
        ],
        scratch_types=[
            pltpu.VMEM((TPW,), jnp.int32),
            pltpu.VMEM((TPW,), jnp.int32),
            pltpu.VMEM((TPW,), jnp.float32),
            pltpu.VMEM((TPW,), jnp.float32),
            pltpu.VMEM((TPW, H), jnp.float32),
            pltpu.SemaphoreType.DMA,
        ],
    )
    def k(x_hbm, p0_hbm, p1_hbm, w0_hbm, w1_hbm, xs_hbm, ws_hbm,
          i0_v, i1_v, w0_v, w1_v, rows_v, sem):
        wid = lax.axis_index("s") * 2 + lax.axis_index("c")
        base = wid * TPW
        loads = [
            pltpu.async_copy(p0_hbm.at[pl.ds(base, TPW)], i0_v, sem),
            pltpu.async_copy(p1_hbm.at[pl.ds(base, TPW)], i1_v, sem),
            pltpu.async_copy(w0_hbm.at[pl.ds(base, TPW)], w0_v, sem),
            pltpu.async_copy(w1_hbm.at[pl.ds(base, TPW)], w1_v, sem),
            pltpu.async_copy(x_hbm.at[pl.ds(base, TPW)], rows_v, sem),
        ]
        for cp in loads:
            cp.wait()
        stores = [
            pltpu.async_copy(rows_v, xs_hbm.at[i0_v], sem),
            pltpu.async_copy(rows_v, xs_hbm.at[i1_v], sem),
            pltpu.async_copy(w0_v, ws_hbm.at[i0_v], sem),
            pltpu.async_copy(w1_v, ws_hbm.at[i1_v], sem),
        ]
        for cp in stores:
            cp.wait()

    return k(x, pos0, pos1, w0, w1)


def _gmm_body(starts_ref, xs_ref, ws_ref, wgu_ref, bgu_ref, wd_ref, bd_ref,
              ys_ref):
    b = pl.program_id(0)
    row0 = b * CB
    xb = xs_ref[...].astype(jnp.bfloat16)
    riota = jax.lax.broadcasted_iota(jnp.int32, (CB, 1), 0)
    ys_ref[...] = jnp.zeros_like(ys_ref)
    for e in range(E):
        lo = jnp.maximum(starts_ref[0, e], row0)
        hi = jnp.minimum(starts_ref[0, e + 1], row0 + CB)

        @pl.when(hi > lo)
        def _(e=e, lo=lo, hi=hi):
            gu = jnp.dot(xb, wgu_ref[e], preferred_element_type=jnp.float32)
            gu = gu + bgu_ref[e, 0][None, :]
            gate = jnp.minimum(gu[:, :INTER], LIMIT)
            up = jnp.clip(gu[:, INTER:], -LIMIT, LIMIT)
            glu = gate * jax.nn.sigmoid(gate * ALPHA)
            act = ((up + 1.0) * glu).astype(jnp.bfloat16)
            dn = jnp.dot(act, wd_ref[e], preferred_element_type=jnp.float32)
            dn = dn + bd_ref[e, 0][None, :]
            msk = (riota >= lo - row0) & (riota < hi - row0)
            ys_ref[...] += jnp.where(msk, dn, 0.0)
    ys_ref[...] *= ws_ref[...]


def _run_gmm(starts, xs, ws, wgu, bgu, wd, bd):
    return pl.pallas_call(
        _gmm_body,
        grid=(NSLOT // CB,),
        in_specs=[
            pl.BlockSpec(memory_space=pltpu.SMEM),
            pl.BlockSpec((CB, H), lambda b: (b, 0)),
            pl.BlockSpec((CB, 1), lambda b: (b, 0)),
            pl.BlockSpec((E, H, 2 * INTER), lambda b: (0, 0, 0)),
            pl.BlockSpec((E, 1, 2 * INTER), lambda b: (0, 0, 0)),
            pl.BlockSpec((E, INTER, H), lambda b: (0, 0, 0)),
            pl.BlockSpec((E, 1, H), lambda b: (0, 0, 0)),
        ],
        out_specs=pl.BlockSpec((CB, H), lambda b: (b, 0)),
        out_shape=jax.ShapeDtypeStruct((NSLOT, H), jnp.float32),
    )(starts, xs, ws, wgu, bgu, wd, bd)


def _combine_sc(ys, pos0, pos1):
    mesh = plsc.VectorSubcoreMesh(core_axis_name="c", subcore_axis_name="s")

    @functools.partial(
        pl.kernel, mesh=mesh,
        out_type=jax.ShapeDtypeStruct((S, H), jnp.float32),
        scratch_types=[
            pltpu.VMEM((TPW,), jnp.int32),
            pltpu.VMEM((TPW,), jnp.int32),
            pltpu.VMEM((TPW, H), jnp.float32),
            pltpu.VMEM((TPW, H), jnp.float32),
            pltpu.SemaphoreType.DMA,
        ],
    )
    def k(ys_hbm, p0_hbm, p1_hbm, out_hbm, i0_v, i1_v, bufa, bufb, sem):
        wid = lax.axis_index("s") * 2 + lax.axis_index("c")
        base = wid * TPW
        l0 = pltpu.async_copy(p0_hbm.at[pl.ds(base, TPW)], i0_v, sem)
        l1 = pltpu.async_copy(p1_hbm.at[pl.ds(base, TPW)], i1_v, sem)
        l0.wait()
        l1.wait()
        g0 = pltpu.async_copy(ys_hbm.at[i0_v], bufa, sem)
        g1 = pltpu.async_copy(ys_hbm.at[i1_v], bufb, sem)
        g0.wait()
        g1.wait()

        def addrow(r, _):
            for cc in range(H // 16):
                sl = pl.ds(cc * 16, 16)
                bufa[r, sl] += bufb[r, sl]
            return 0
        lax.fori_loop(0, TPW, addrow, 0)
        pltpu.sync_copy(bufa, out_hbm.at[pl.ds(base, TPW)])

    return k(ys, pos0, pos1)


def kernel(hidden_states, router_weight, router_bias, gate_up_proj, gate_up_bias,
           down_proj, down_bias):
    B = hidden_states.shape[0]
    x = hidden_states.reshape(S, H)
    xb = x.astype(jnp.bfloat16)
    wr = router_weight.astype(jnp.bfloat16)
    rb = router_bias.reshape(1, E)
    wgu = gate_up_proj.astype(jnp.bfloat16)
    wd = down_proj.astype(jnp.bfloat16)
    bgu = gate_up_bias.reshape(E, 1, 2 * INTER)
    bd = down_bias.reshape(E, 1, H)

    idx2, w2, starts = _run_router(xb, wr, rb)
    pos0, pos1 = idx2[0], idx2[1]
    xs, ws = _dispatch_sc(x, pos0, pos1, w2[0], w2[1])
    ys = _run_gmm(starts, xs, ws.reshape(NSLOT, 1), wgu, bgu, wd, bd)
    out = _combine_sc(ys, pos0, pos1)
    return out.reshape(B, S, H)
